# Initial kernel scaffold; baseline (speedup 1.0000x reference)
import jax
import jax.numpy as jnp
from jax.experimental import pallas as pl


def kernel(c, cs, deltas, noise_eps):
    raise NotImplementedError("write your pallas kernel here")


# speedup vs baseline: 10.0238x; 10.0238x over previous
import functools

import jax
import jax.numpy as jnp
from jax import lax
from jax.experimental import pallas as pl
from jax.experimental.pallas import tpu as pltpu
from jax.experimental.pallas import tpu_sc as plsc

N = 1048576
M = 4194304
K = 64
R = M // K
NW = 32
QT = N // NW
CB = 128
NCHUNK = QT // CB
L = 16
NG = CB // L


def _count_lt(table_ref, idx_b, c_vec, n, pos0):
    pos = pos0
    s = n // 2
    while s >= 1:
        probe_at = pos + (s - 1)
        if idx_b is None:
            probe = plsc.load_gather(table_ref, [probe_at])
        else:
            probe = plsc.load_gather(table_ref, [idx_b, probe_at])
        pos = pos + jnp.where(probe < c_vec, s, 0).astype(jnp.int32)
        s //= 2
    if idx_b is None:
        probe = plsc.load_gather(table_ref, [pos])
    else:
        probe = plsc.load_gather(table_ref, [idx_b, pos])
    return pos + (probe < c_vec).astype(jnp.int32)


def _body(c_hbm, t0_hbm, cs2d_hbm, tail_hbm, eps_hbm, out_hbm,
          t0_v, c_v, n_v, out_v, ridx_v, rowsA_v, rowsB_v,
          semA, semB):
    wid = lax.axis_index("s") * 2 + lax.axis_index("c")
    base = wid * QT

    pltpu.sync_copy(t0_hbm, t0_v)

    iota = lax.iota(jnp.int32, L)
    zeros16 = jnp.zeros((L,), jnp.int32)

    def chunk(i, carry):
        cbase = base + i * CB
        pltpu.sync_copy(c_hbm.at[pl.ds(cbase, CB)], c_v)

        for g in range(NG):
            c_vec = c_v[pl.ds(g * L, L)]
            ct = _count_lt(t0_v, None, c_vec, R, zeros16)
            ridx_v[pl.ds(g * L, L)] = jnp.maximum(ct - 1, 0)

        copyA = pltpu.make_async_copy(cs2d_hbm.at[ridx_v], rowsA_v, semA)
        copyA.start()
        copyB = pltpu.make_async_copy(tail_hbm.at[ridx_v], rowsB_v, semB)
        copyB.start()
        pltpu.sync_copy(eps_hbm.at[pl.ds(cbase, CB)], n_v)
        copyA.wait()
        copyB.wait()

        for g in range(NG):
            b = iota + g * L
            c_vec = c_v[pl.ds(g * L, L)]
            eps = n_v[pl.ds(g * L, L)]
            r = ridx_v[pl.ds(g * L, L)]
            o = _count_lt(rowsA_v, b, c_vec, K, zeros16)
            ind = K * r + o
            ind_c = jnp.minimum(ind, M - 1)
            o_c = ind_c - K * r
            wl = plsc.load_gather(rowsA_v, [b, jnp.maximum(o_c - 1, 0)])
            am = plsc.load_gather(rowsA_v, [b, jnp.minimum(o_c, K - 1)])
            bm = plsc.load_gather(rowsB_v, [b, jnp.clip(o_c - K, 0, 15)])
            wm = jnp.where(o_c < K, am, bm)
            jh = o_c + 1
            ah = plsc.load_gather(rowsA_v, [b, jnp.minimum(jh, K - 1)])
            bh = plsc.load_gather(rowsB_v, [b, jnp.clip(jh - K, 0, 15)])
            wh = jnp.where(jh < K, ah, bh)
            zf = jnp.zeros((L,), jnp.float32)
            dlo = jnp.where(ind_c >= 1, wm - wl, zf)
            dhi = jnp.where(ind_c <= M - 2, wh - wm, zf)
            delta = jnp.maximum(dlo, dhi)
            out_v[pl.ds(g * L, L)] = c_vec + 0.5 * delta * eps

        pltpu.sync_copy(out_v, out_hbm.at[pl.ds(cbase, CB)])
        return carry

    lax.fori_loop(0, NCHUNK, chunk, 0)


@jax.jit
def kernel(c, cs, deltas, noise_eps):
    del deltas
    mesh = plsc.VectorSubcoreMesh(core_axis_name="c", subcore_axis_name="s")
    run = pl.kernel(
        _body,
        out_type=jax.ShapeDtypeStruct((N,), jnp.float32),
        mesh=mesh,
        scratch_types=[
            pltpu.VMEM((R,), jnp.float32),
            pltpu.VMEM((CB,), jnp.float32),
            pltpu.VMEM((CB,), jnp.float32),
            pltpu.VMEM((CB,), jnp.float32),
            pltpu.VMEM((CB,), jnp.int32),
            pltpu.VMEM((CB, K), jnp.float32),
            pltpu.VMEM((CB, 16), jnp.float32),
            pltpu.SemaphoreType.DMA,
            pltpu.SemaphoreType.DMA,
        ],
        compiler_params=pltpu.CompilerParams(
            needs_layout_passes=False, use_tc_tiling_on_sc=False),
    )
    tail_tab = jnp.concatenate([cs[K:], cs[:K]]).reshape(R, K)[:, :16]
    out = run(c.reshape(-1), cs[::K], cs.reshape(R, K),
              tail_tab, noise_eps.reshape(-1))
    return out.reshape(c.shape)


# speedup vs baseline: 23.5381x; 2.3482x over previous
import jax
import jax.numpy as jnp
from jax import lax
from jax.experimental import pallas as pl
from jax.experimental.pallas import tpu as pltpu
from jax.experimental.pallas import tpu_sc as plsc

N = 1048576
M = 4194304
K = 64
R = M // K
NW = 32
QT = N // NW
CB = 128
NCHUNK = QT // CB
L = 16
NG = CB // L


def _count_lt_multi(load, c_vecs, n):
    pos = [jnp.zeros((L,), jnp.int32)] * NG
    s = n // 2
    while s >= 1:
        probes = [load(g, pos[g] + (s - 1)) for g in range(NG)]
        pos = [pos[g] + jnp.where(probes[g] < c_vecs[g], s, 0).astype(jnp.int32)
               for g in range(NG)]
        s //= 2
    probes = [load(g, pos[g]) for g in range(NG)]
    return [pos[g] + (probes[g] < c_vecs[g]).astype(jnp.int32)
            for g in range(NG)]


def _body(c_hbm, t0_hbm, cs2d_hbm, tail_hbm, eps_hbm, out_hbm,
          t0_v, c_v, n_v, out_v, ridx_v, rowsA_v, rowsB_v,
          semC, semA, semB, semE, semO):
    wid = lax.axis_index("s") * 2 + lax.axis_index("c")
    base = wid * QT
    iota = lax.iota(jnp.int32, L)

    def c_copy(i, cslot):
        cbase = base + i * CB
        return pltpu.make_async_copy(
            c_hbm.at[pl.ds(cbase, CB)], c_v.at[cslot], semC.at[cslot])

    def gatherA(b):
        return pltpu.make_async_copy(
            cs2d_hbm.at[ridx_v.at[b]], rowsA_v.at[b], semA.at[b])

    def gatherB(b):
        return pltpu.make_async_copy(
            tail_hbm.at[ridx_v.at[b]], rowsB_v.at[b], semB.at[b])

    def eps_copy(i, b):
        cbase = base + i * CB
        return pltpu.make_async_copy(
            eps_hbm.at[pl.ds(cbase, CB)], n_v.at[b], semE.at[b])

    def out_copy(i, b):
        cbase = base + i * CB
        return pltpu.make_async_copy(
            out_v.at[b], out_hbm.at[pl.ds(cbase, CB)], semO.at[b])

    def coarse(i, b, cslot):
        if isinstance(i, int):
            if i + 2 < NCHUNK:
                c_copy(i + 2, (cslot + 2) % 4).start()
        else:
            @pl.when(i + 2 < NCHUNK)
            def _():
                c_copy(i + 2, (cslot + 2) % 4).start()
        c_copy(i, cslot).wait()
        c_vecs = [c_v[cslot, pl.ds(g * L, L)] for g in range(NG)]
        cts = _count_lt_multi(
            lambda g, p: plsc.load_gather(t0_v, [p]), c_vecs, R)
        for g in range(NG):
            ridx_v[b, pl.ds(g * L, L)] = jnp.maximum(cts[g] - 1, 0)
        gatherA(b).start()
        gatherB(b).start()
        eps_copy(i, b).start()

    def fine(i, b, cslot):
        gatherA(b).wait()
        gatherB(b).wait()
        eps_copy(i, b).wait()

        @pl.when(i >= 2)
        def _():
            out_copy(i, b).wait()
        c_vecs = [c_v[cslot, pl.ds(g * L, L)] for g in range(NG)]
        bidx = [iota + g * L for g in range(NG)]
        os = _count_lt_multi(
            lambda g, p: plsc.load_gather(rowsA_v.at[b], [bidx[g], p]),
            c_vecs, K)
        for g in range(NG):
            b_g = bidx[g]
            r = ridx_v[b, pl.ds(g * L, L)]
            ind = K * r + os[g]
            ind_c = jnp.minimum(ind, M - 1)
            o_c = ind_c - K * r
            wl = plsc.load_gather(rowsA_v.at[b], [b_g, jnp.maximum(o_c - 1, 0)])
            am = plsc.load_gather(rowsA_v.at[b], [b_g, jnp.minimum(o_c, K - 1)])
            bm = plsc.load_gather(rowsB_v.at[b], [b_g, jnp.clip(o_c - K, 0, 15)])
            wm = jnp.where(o_c < K, am, bm)
            jh = o_c + 1
            ah = plsc.load_gather(rowsA_v.at[b], [b_g, jnp.minimum(jh, K - 1)])
            bh = plsc.load_gather(rowsB_v.at[b], [b_g, jnp.clip(jh - K, 0, 15)])
            wh = jnp.where(jh < K, ah, bh)
            zf = jnp.zeros((L,), jnp.float32)
            dlo = jnp.where(ind_c >= 1, wm - wl, zf)
            dhi = jnp.where(ind_c <= M - 2, wh - wm, zf)
            delta = jnp.maximum(dlo, dhi)
            eps = n_v[b, pl.ds(g * L, L)]
            out_v[b, pl.ds(g * L, L)] = c_vecs[g] + 0.5 * delta * eps
        out_copy(i, b).start()

    pltpu.sync_copy(t0_hbm, t0_v)

    c_copy(0, 0).start()
    c_copy(1, 1).start()
    coarse(0, 0, 0)

    def quad(q, carry):
        i0 = 4 * q
        coarse(i0 + 1, 1, 1)
        fine(i0, 0, 0)
        coarse(i0 + 2, 0, 2)
        fine(i0 + 1, 1, 1)
        coarse(i0 + 3, 1, 3)
        fine(i0 + 2, 0, 2)

        @pl.when(i0 + 4 < NCHUNK)
        def _():
            coarse(i0 + 4, 0, 0)
        fine(i0 + 3, 1, 3)
        return carry

    lax.fori_loop(0, NCHUNK // 4, quad, 0)
    out_copy(NCHUNK - 2, 0).wait()
    out_copy(NCHUNK - 1, 1).wait()


@jax.jit
def kernel(c, cs, deltas, noise_eps):
    del deltas
    mesh = plsc.VectorSubcoreMesh(core_axis_name="c", subcore_axis_name="s")
    run = pl.kernel(
        _body,
        out_type=jax.ShapeDtypeStruct((N,), jnp.float32),
        mesh=mesh,
        scratch_types=[
            pltpu.VMEM((R,), jnp.float32),
            pltpu.VMEM((4, CB), jnp.float32),
            pltpu.VMEM((2, CB), jnp.float32),
            pltpu.VMEM((2, CB), jnp.float32),
            pltpu.VMEM((2, CB), jnp.int32),
            pltpu.VMEM((2, CB, K), jnp.float32),
            pltpu.VMEM((2, CB, 16), jnp.float32),
            pltpu.SemaphoreType.DMA((4,)),
            pltpu.SemaphoreType.DMA((2,)),
            pltpu.SemaphoreType.DMA((2,)),
            pltpu.SemaphoreType.DMA((2,)),
            pltpu.SemaphoreType.DMA((2,)),
        ],
        compiler_params=pltpu.CompilerParams(
            needs_layout_passes=False, use_tc_tiling_on_sc=False),
    )
    tail_tab = jnp.concatenate([cs[K:], cs[:K]]).reshape(R, K)[:, :16]
    out = run(c.reshape(-1), cs[::K], cs.reshape(R, K),
              tail_tab, noise_eps.reshape(-1))
    return out.reshape(c.shape)
